# Initial kernel scaffold; baseline (speedup 1.0000x reference)
import jax
import jax.numpy as jnp
from jax.experimental import pallas as pl


def kernel(x, edge_index, super_nodes):
    raise NotImplementedError("write your pallas kernel here")


# speedup vs baseline: 1.6421x; 1.6421x over previous
import functools

import jax
import jax.numpy as jnp
from jax import lax
from jax.experimental import pallas as pl
from jax.experimental.pallas import tpu as pltpu
from jax.experimental.pallas import tpu_sc as plsc

D = 128
P = 256
N = 4096
NT = N + P
E = 65536

BLK = 256
NBLK = NT // BLK

NC = 2
NS = 16
L = 16
NW = NC * NS
CH = E // NW
ROWS = CH // 128


def _dense_body(f_ref, ft_ref, o_ref):
    i = pl.program_id(0)
    f = f_ref[...]
    right = lax.dot_general(f, ft_ref[:, N:],
                            (((1,), (0,)), ((), ())),
                            preferred_element_type=jnp.float32)
    o_ref[:, N:] = jax.nn.sigmoid(right)

    @pl.when(i < NBLK - 1)
    def _zero():
        o_ref[:, :N] = jnp.zeros((BLK, N), jnp.float32)

    @pl.when(i == NBLK - 1)
    def _bottom():
        left = lax.dot_general(f, ft_ref[:, :N],
                               (((1,), (0,)), ((), ())),
                               preferred_element_type=jnp.float32)
        o_ref[:, :N] = jax.nn.sigmoid(left)


_dense = pl.pallas_call(
    _dense_body,
    grid=(NBLK,),
    in_specs=[
        pl.BlockSpec((BLK, D), lambda i: (i, 0)),
        pl.BlockSpec((D, NT), lambda i: (0, 0)),
    ],
    out_specs=pl.BlockSpec((BLK, NT), lambda i: (i, 0)),
    out_shape=jax.ShapeDtypeStruct((NT, NT), jnp.float32),
    compiler_params=pltpu.CompilerParams(
        dimension_semantics=("arbitrary",)),
)


@functools.partial(
    pl.kernel,
    mesh=plsc.VectorSubcoreMesh(core_axis_name="c", subcore_axis_name="s"),
    scratch_types=[
        pltpu.VMEM((CH,), jnp.int32),
        pltpu.VMEM((CH,), jnp.int32),
        pltpu.VMEM((ROWS, 128), jnp.int32),
        pltpu.VMEM((ROWS, 128), jnp.float32),
        pltpu.SemaphoreType.DMA,
    ],
)
def _scatter(src_hbm, dst_hbm, out_hbm, src_v, dst_v, idx_v, ones_v, sem):
    wid = lax.axis_index("s") * NC + lax.axis_index("c")
    base = wid * CH
    pltpu.sync_copy(src_hbm.at[pl.ds(base, CH)], src_v)
    pltpu.sync_copy(dst_hbm.at[pl.ds(base, CH)], dst_v)
    for k in range(CH // L):
        j, c = divmod(k, 128 // L)
        s = src_v[pl.ds(k * L, L)]
        t = dst_v[pl.ds(k * L, L)]
        idx_v[j, pl.ds(c * L, L)] = s * NT + t
        ones_v[j, pl.ds(c * L, L)] = jnp.ones((L,), jnp.float32)
    copies = [
        pltpu.async_copy(ones_v.at[j], out_hbm.at[idx_v.at[j]], sem)
        for j in range(ROWS)
    ]
    for cp in copies:
        cp.wait()


def kernel(x, edge_index, super_nodes):
    f = jnp.concatenate([x, super_nodes], axis=0)
    ft = f.T
    dense = _dense(f, ft)
    src = edge_index[0].astype(jnp.int32)
    dst = edge_index[1].astype(jnp.int32)
    out_ref = jax.new_ref(dense.reshape(NT * NT))
    _scatter(src, dst, out_ref)
    return out_ref[...].reshape(NT, NT)


# speedup vs baseline: 3.5826x; 2.1817x over previous
import functools

import jax
import jax.numpy as jnp
from jax import lax
from jax.experimental import pallas as pl
from jax.experimental.pallas import tpu as pltpu
from jax.experimental.pallas import tpu_sc as plsc

D = 128
P = 256
N = 4096
NT = N + P
E = 65536

BLK = 256
NBLK = NT // BLK

NC = 2
NS = 16
L = 16
NW = NC * NS
CH = E // NW
ROWS = CH // 128


def _dense_body(f_ref, ft_ref, o_ref):
    i = pl.program_id(0)
    f = f_ref[...]
    right = lax.dot_general(f, ft_ref[:, N:],
                            (((1,), (0,)), ((), ())),
                            preferred_element_type=jnp.float32)
    o_ref[:, N:] = jax.nn.sigmoid(right)

    @pl.when(i < NBLK - 1)
    def _zero():
        o_ref[:, :N] = jnp.zeros((BLK, N), jnp.float32)

    @pl.when(i == NBLK - 1)
    def _bottom():
        left = lax.dot_general(f, ft_ref[:, :N],
                               (((1,), (0,)), ((), ())),
                               preferred_element_type=jnp.float32)
        o_ref[:, :N] = jax.nn.sigmoid(left)


_dense = pl.pallas_call(
    _dense_body,
    grid=(NBLK,),
    in_specs=[
        pl.BlockSpec((BLK, D), lambda i: (i, 0)),
        pl.BlockSpec((D, NT), lambda i: (0, 0)),
    ],
    out_specs=pl.BlockSpec((BLK, NT), lambda i: (i, 0)),
    out_shape=jax.ShapeDtypeStruct((NT, NT), jnp.float32),
    compiler_params=pltpu.CompilerParams(
        dimension_semantics=("arbitrary",)),
)


@functools.partial(
    pl.kernel,
    mesh=plsc.VectorSubcoreMesh(core_axis_name="c", subcore_axis_name="s"),
    scratch_types=[
        pltpu.VMEM((CH,), jnp.int32),
        pltpu.VMEM((CH,), jnp.int32),
        pltpu.VMEM((ROWS, 128), jnp.int32),
        pltpu.VMEM((ROWS, 128), jnp.float32),
        pltpu.SemaphoreType.DMA,
    ],
)
def _scatter(src_hbm, dst_hbm, out_hbm, src_v, dst_v, idx_v, ones_v, sem):
    wid = lax.axis_index("s") * NC + lax.axis_index("c")
    base = wid * CH
    pltpu.sync_copy(src_hbm.at[pl.ds(base, CH)], src_v)
    pltpu.sync_copy(dst_hbm.at[pl.ds(base, CH)], dst_v)
    for k in range(CH // L):
        j, c = divmod(k, 128 // L)
        s = src_v[pl.ds(k * L, L)]
        t = dst_v[pl.ds(k * L, L)]
        flat = (
            ((s >> 3) * (34 * 1024) + (t >> 7) * 1024)
            + ((s & 7) << 7)
            + (t & 127)
        )
        idx_v[j, pl.ds(c * L, L)] = flat
        ones_v[j, pl.ds(c * L, L)] = jnp.ones((L,), jnp.float32)
    copies = [
        pltpu.async_copy(ones_v.at[j], out_hbm.at[idx_v.at[j]], sem)
        for j in range(ROWS)
    ]
    for cp in copies:
        cp.wait()


def kernel(x, edge_index, super_nodes):
    f = jnp.concatenate([x, super_nodes], axis=0)
    ft = f.T
    dense = _dense(f, ft)
    src = edge_index[0].astype(jnp.int32)
    dst = edge_index[1].astype(jnp.int32)
    d4 = dense.reshape(NT // 8, 8, NT // 128, 128).transpose(0, 2, 1, 3)
    out_ref = jax.new_ref(d4.reshape(NT * NT))
    _scatter(src, dst, out_ref)
    out4 = out_ref[...].reshape(NT // 8, NT // 128, 8, 128)
    return out4.transpose(0, 2, 1, 3).reshape(NT, NT)
